# baseline (speedup 1.0000x reference)
import jax
import jax.numpy as jnp
from jax import lax
from jax.experimental import pallas as pl
from jax.experimental.pallas import tpu as pltpu
from jax.experimental.pallas import tpu_sc as plsc

NUM_CLASSES = 1000000
HIDDEN = 64
BATCH = 16384

_info = plsc.get_sparse_core_info()
_NC, _NS, _L = _info.num_cores, _info.num_subcores, _info.num_lanes
_NW = _NC * _NS
_SPAN = 31256
_R = 320
_NCH = -(-_SPAN // _R)
_CLAMP = NUM_CLASSES - _R
_PIECE = 2048
_H = 768
_CCAP = 64
_SROWS = 128
_FLUSH = 64
_DUMP = BATCH
_OUT_ROWS = BATCH + 48


def _body(labels_hbm, table_hbm, out_hbm, chunk0_v, chunk1_v, stage_v,
          sdest_v, piece_v, hrow_v, hdst_v, clist_r, clist_d, tmp_v,
          sem, sem_out):
    wid = lax.axis_index("s") * _NC + lax.axis_index("c")
    lo = wid * _SPAN
    hi = lo + _SPAN
    iota = lax.iota(jnp.int32, _L)
    dumpv = jnp.full((_L,), _DUMP, jnp.int32)

    def chunk_start(k):
        return jnp.minimum(lo + k * _R, _CLAMP)

    def fire(k):
        def f0(_):
            pltpu.async_copy(
                table_hbm.at[pl.ds(chunk_start(k), _R), :], chunk0_v, sem)
            return 0

        def f1(_):
            pltpu.async_copy(
                table_hbm.at[pl.ds(chunk_start(k), _R), :], chunk1_v, sem)
            return 0

        lax.cond(k % 2 == 0, f0, f1, 0)

    pltpu.async_copy(table_hbm.at[pl.ds(chunk_start(0), _R), :], chunk0_v, sem)
    pltpu.async_copy(table_hbm.at[pl.ds(chunk_start(1), _R), :], chunk1_v, sem)

    nhit = 0
    for p in range(BATCH // _PIECE):
        pltpu.sync_copy(labels_hbm.at[pl.ds(p * _PIECE, _PIECE)], piece_v)

        def piece_group(g, off, p=p):
            v = piece_v[pl.ds(g * _L, _L)]
            m = (v >= lo) & (v < hi)
            pc = jnp.sum(m.astype(jnp.int32))

            @pl.when(pc > 0)
            def _():
                o = jnp.minimum(off, _H - _L)
                plsc.store_compressed(tmp_v.at[pl.ds(0, _L)], v, mask=m)
                plsc.store_compressed(
                    tmp_v.at[pl.ds(_L, _L)], p * _PIECE + g * _L + iota,
                    mask=m)
                hrow_v[pl.ds(o, _L)] = tmp_v[pl.ds(0, _L)]
                hdst_v[pl.ds(o, _L)] = tmp_v[pl.ds(_L, _L)]

            return off + pc

        nhit = lax.fori_loop(0, _PIECE // _L, piece_group, nhit)
    nhit = jnp.minimum(nhit, _H)
    nhgrp = (nhit + _L - 1) // _L

    for s in range(_SROWS // _L):
        sdest_v[0, pl.ds(s * _L, _L)] = dumpv

    def flush(u):
        def do_flush(_):
            pltpu.async_copy(stage_v, out_hbm.at[sdest_v.at[0]], sem_out).wait()
            for s in range(_SROWS // _L):
                sdest_v[0, pl.ds(s * _L, _L)] = dumpv
            return 0

        return lax.cond(u >= _FLUSH, do_flush, lambda _: u, 0)

    def do_chunk(k, used):
        pltpu.make_async_copy(
            table_hbm.at[pl.ds(0, _R), :], chunk0_v, sem).wait()
        c0 = chunk_start(k)

        def refilt(g, nc):
            r = hrow_v[pl.ds(g * _L, _L)]
            m = (r >= c0) & (r < c0 + _R) & (g * _L + iota < nhit)
            pc = jnp.sum(m.astype(jnp.int32))

            @pl.when(pc > 0)
            def _():
                o = jnp.minimum(nc, _CCAP - _L)
                plsc.store_compressed(tmp_v.at[pl.ds(0, _L)], r, mask=m)
                plsc.store_compressed(
                    tmp_v.at[pl.ds(_L, _L)], hdst_v[pl.ds(g * _L, _L)],
                    mask=m)
                clist_r[pl.ds(o, _L)] = tmp_v[pl.ds(0, _L)]
                clist_d[pl.ds(o, _L)] = tmp_v[pl.ds(_L, _L)]

            return nc + pc

        nc = lax.fori_loop(0, nhgrp, refilt, 0)
        nc = jnp.minimum(nc, _CCAP - _L)
        clist_r[pl.ds(nc, _L)] = iota * 0 + c0
        clist_d[pl.ds(nc, _L)] = dumpv
        ng = (nc + _L - 1) // _L

        def extract_from(chunk_ref, u):
            def ex(q, _):
                rv = clist_r[pl.ds(q * _L, _L)] - c0
                dv = clist_d[pl.ds(q * _L, _L)]
                sdest_v[0, pl.ds(u + q * _L, _L)] = dv
                for j in range(_L):
                    r = rv[j]
                    su = u + q * _L + j
                    for c in range(HIDDEN // _L):
                        stage_v[su, pl.ds(c * _L, _L)] = (
                            chunk_ref[r, pl.ds(c * _L, _L)])
                return 0

            lax.fori_loop(0, ng, ex, 0)
            return 0

        lax.cond(k % 2 == 0,
                 lambda u: extract_from(chunk0_v, u),
                 lambda u: extract_from(chunk1_v, u),
                 used)

        @pl.when(k + 2 < _NCH)
        def _():
            fire(k + 2)

        return flush(used + ng * _L)

    used = lax.fori_loop(0, _NCH, do_chunk, 0)
    pltpu.async_copy(stage_v, out_hbm.at[sdest_v.at[0]], sem_out).wait()


@jax.jit
def kernel(labels, table):
    f = pl.kernel(
        _body,
        mesh=plsc.VectorSubcoreMesh(core_axis_name="c", subcore_axis_name="s"),
        out_type=jax.ShapeDtypeStruct((_OUT_ROWS, 2 * HIDDEN), jnp.float32),
        scratch_types=[
            pltpu.VMEM((_R, HIDDEN), jnp.float32),
            pltpu.VMEM((_R, HIDDEN), jnp.float32),
            pltpu.VMEM((_SROWS, 2 * HIDDEN), jnp.float32),
            pltpu.VMEM((1, _SROWS), jnp.int32),
            pltpu.VMEM((_PIECE,), jnp.int32),
            pltpu.VMEM((_H,), jnp.int32),
            pltpu.VMEM((_H,), jnp.int32),
            pltpu.VMEM((_CCAP,), jnp.int32),
            pltpu.VMEM((_CCAP,), jnp.int32),
            pltpu.VMEM((2 * _L,), jnp.int32),
            pltpu.SemaphoreType.DMA,
            pltpu.SemaphoreType.DMA,
        ],
        compiler_params=pltpu.CompilerParams(needs_layout_passes=False),
    )
    out_full = f(labels.astype(jnp.int32), table)
    return out_full[:BATCH, :HIDDEN]


# speedup vs baseline: 11.9947x; 11.9947x over previous
import jax
import jax.numpy as jnp
from jax import lax
from jax.experimental import pallas as pl
from jax.experimental.pallas import tpu as pltpu
from jax.experimental.pallas import tpu_sc as plsc

NUM_CLASSES = 1000000
HIDDEN = 64
BATCH = 16384

_info = plsc.get_sparse_core_info()
_NC, _NS, _L = _info.num_cores, _info.num_subcores, _info.num_lanes
_NW = _NC * _NS
_BPW = BATCH // _NW


_NSEM = 8


def _gather_body(labels_hbm, table_hbm, out_hbm, lab_v, out_v, *sems):
    wid = lax.axis_index("s") * _NC + lax.axis_index("c")
    base = wid * _BPW
    pltpu.sync_copy(labels_hbm.at[pl.ds(base, _BPW)], lab_v)

    def fire(g, _):
        v = lab_v[pl.ds(g * _L, _L)]
        for j in range(_L):
            pltpu.async_copy(
                table_hbm.at[pl.ds(v[j], 1), :],
                out_v.at[pl.ds(g * _L + j, 1), :],
                sems[j % _NSEM],
            )
        return 0

    lax.fori_loop(0, _BPW // _L, fire, 0)

    def drain(i, _):
        for j in range(_NSEM):
            pltpu.make_async_copy(
                table_hbm.at[pl.ds(0, 1), :],
                out_v.at[pl.ds(i * _NSEM + j, 1), :],
                sems[j],
            ).wait()
        return 0

    lax.fori_loop(0, _BPW // _NSEM, drain, 0)
    pltpu.sync_copy(out_v, out_hbm.at[pl.ds(base, _BPW)])


@jax.jit
def kernel(labels, table):
    f = pl.kernel(
        _gather_body,
        mesh=plsc.VectorSubcoreMesh(core_axis_name="c", subcore_axis_name="s"),
        out_type=jax.ShapeDtypeStruct((BATCH, HIDDEN), jnp.float32),
        scratch_types=[
            pltpu.VMEM((_BPW,), jnp.int32),
            pltpu.VMEM((_BPW, HIDDEN), jnp.float32),
        ] + [pltpu.SemaphoreType.DMA] * _NSEM,
    )
    return f(labels.astype(jnp.int32), table)
